# baseline (speedup 1.0000x reference)
import functools

import jax
import jax.numpy as jnp
from jax import lax
from jax.experimental import pallas as pl
from jax.experimental.pallas import tpu as pltpu
from jax.experimental.pallas import tpu_sc as plsc

NUM_TYPES = 1000
SEQ_LEN = 3276800
NC = 2
NS = 16
NW = NC * NS
CHUNK = SEQ_LEN // NW
BLK = 10240
NBLK = CHUNK // BLK
ZSLOT = NUM_TYPES * NUM_TYPES
WPAD = ZSLOT + 16


def _body(seq_h, w_h, pad_h, out_h,
          w_sh, buf0, buf1, idx0, idx1, val0, val1, padv,
          seq_sem, gat_sem, w_sem):
    c = lax.axis_index("c")
    s = lax.axis_index("s")
    wid = s * NC + c
    base = wid * CHUNK
    bufs, idxs, vals = (buf0, buf1), (idx0, idx1), (val0, val1)
    islast = wid == NW - 1

    @pl.when(s == 0)
    def _():
        pltpu.make_async_copy(w_h, w_sh, w_sem).start()

    pltpu.sync_copy(pad_h, padv)
    pad = padv[...]

    def issue_seq(j):
        b = bufs[j % 2]
        o = base + j * BLK
        if j < NBLK - 1:
            pltpu.make_async_copy(seq_h.at[pl.ds(o, BLK + 16)], b,
                                  seq_sem).start()
        else:
            @pl.when(islast)
            def _():
                pltpu.make_async_copy(seq_h.at[pl.ds(o, BLK)],
                                      b.at[pl.ds(0, BLK)], seq_sem).start()

            @pl.when(jnp.logical_not(islast))
            def _():
                pltpu.make_async_copy(seq_h.at[pl.ds(o, BLK + 16)], b,
                                      seq_sem).start()

    def wait_seq(j):
        b = bufs[j % 2]
        o = base + j * BLK
        if j < NBLK - 1:
            pltpu.make_async_copy(seq_h.at[pl.ds(o, BLK + 16)], b,
                                  seq_sem).wait()
        else:
            @pl.when(islast)
            def _():
                pltpu.make_async_copy(seq_h.at[pl.ds(o, BLK)],
                                      b.at[pl.ds(0, BLK)], seq_sem).wait()
                b[pl.ds(BLK, 16)] = pad

            @pl.when(jnp.logical_not(islast))
            def _():
                pltpu.make_async_copy(seq_h.at[pl.ds(o, BLK + 16)], b,
                                      seq_sem).wait()

    def gather(j):
        return pltpu.make_async_copy(w_sh.at[idxs[j % 2]], vals[j % 2],
                                     gat_sem)

    def merged(j, acc, accumulate):
        b, ij = bufs[j % 2], idxs[j % 2]
        vprev = vals[j % 2]

        @plsc.parallel_loop(0, BLK, step=16, unroll=8, carry=acc)
        def out(i, a3):
            a = b[pl.ds(i, 16)]
            nxt = b[pl.ds(i + 1, 16)]
            ij[pl.ds(i, 16)] = a * NUM_TYPES + nxt
            if accumulate:
                a3 = a3 + vprev[pl.ds(i, 16)]
            return a3

        return out

    def accum_tail2(acc):
        @plsc.parallel_loop(0, BLK, step=16, unroll=8, carry=acc)
        def acc(i, a3):
            return a3 + val0[pl.ds(i, 16)] + val1[pl.ds(i, 16)]

        return acc

    issue_seq(0)
    acc = jnp.zeros((16,), jnp.float32)
    for j in range(NBLK):
        wait_seq(j)
        if j + 1 < NBLK:
            issue_seq(j + 1)
        acc = merged(j, acc, accumulate=(j >= 2))
        if j == 0:
            @pl.when(s == 0)
            def _():
                pltpu.make_async_copy(w_h, w_sh, w_sem).wait()

            plsc.subcore_barrier()
        if j >= 1:
            gather(j - 1).wait()
        gather(j).start()
    gather(NBLK - 1).wait()
    acc = accum_tail2(acc)

    val0[pl.ds(0, 16)] = acc
    pltpu.sync_copy(val0.at[pl.ds(0, 16)], out_h.at[pl.ds(wid * 16, 16)])


@functools.partial(
    pl.kernel,
    out_type=jax.ShapeDtypeStruct((NW * 16,), jnp.float32),
    mesh=plsc.VectorSubcoreMesh(core_axis_name="c", subcore_axis_name="s"),
    scratch_types=[
        pltpu.VMEM_SHARED((ZSLOT,), jnp.float32),
        pltpu.VMEM((BLK + 16,), jnp.int32),
        pltpu.VMEM((BLK + 16,), jnp.int32),
        pltpu.VMEM((BLK,), jnp.int32),
        pltpu.VMEM((BLK,), jnp.int32),
        pltpu.VMEM((BLK,), jnp.float32),
        pltpu.VMEM((BLK,), jnp.float32),
        pltpu.VMEM((16,), jnp.int32),
        pltpu.SemaphoreType.DMA,
        pltpu.SemaphoreType.DMA,
        pltpu.SemaphoreType.DMA,
    ],
)
def _partials(seq_h, w_h, pad_h, out_h, *rest):
    _body(seq_h, w_h, pad_h, out_h, *rest)


def kernel(sequence, padding_idx, W):
    ar = jnp.arange(NUM_TYPES)
    dead = (ar[:, None] == padding_idx) | (ar[None, :] == padding_idx)
    wz = jnp.where(dead, jnp.float32(0), W).reshape(-1)
    padv = jnp.full((16,), padding_idx, dtype=jnp.int32)
    parts = _partials(sequence, wz, padv)
    return -jnp.sum(parts)


# speedup vs baseline: 1.0003x; 1.0003x over previous
import functools

import jax
import jax.numpy as jnp
from jax import lax
from jax.experimental import pallas as pl
from jax.experimental.pallas import tpu as pltpu
from jax.experimental.pallas import tpu_sc as plsc

NUM_TYPES = 1000
SEQ_LEN = 3276800
NC = 2
NS = 16
NW = NC * NS
CHUNK = SEQ_LEN // NW
BLK = 10240
NBLK = CHUNK // BLK
ZSLOT = NUM_TYPES * NUM_TYPES
WPAD = ZSLOT + 16


def _body(seq_h, w_h, pad_h, out_h,
          w_sh, buf0, buf1, idx0, idx1, val0, val1, padv,
          seq_sem, gat_sem, w_sem):
    c = lax.axis_index("c")
    s = lax.axis_index("s")
    wid = s * NC + c
    base = wid * CHUNK
    bufs, idxs, vals = (buf0, buf1), (idx0, idx1), (val0, val1)
    islast = wid == NW - 1

    @pl.when(s == 0)
    def _():
        pltpu.make_async_copy(w_h, w_sh, w_sem).start()

    pltpu.sync_copy(pad_h, padv)
    pad = padv[...]

    def issue_seq(j):
        b = bufs[j % 2]
        o = base + j * BLK
        if j < NBLK - 1:
            pltpu.make_async_copy(seq_h.at[pl.ds(o, BLK + 16)], b,
                                  seq_sem).start()
        else:
            @pl.when(islast)
            def _():
                pltpu.make_async_copy(seq_h.at[pl.ds(o, BLK)],
                                      b.at[pl.ds(0, BLK)], seq_sem).start()

            @pl.when(jnp.logical_not(islast))
            def _():
                pltpu.make_async_copy(seq_h.at[pl.ds(o, BLK + 16)], b,
                                      seq_sem).start()

    def wait_seq(j):
        b = bufs[j % 2]
        o = base + j * BLK
        if j < NBLK - 1:
            pltpu.make_async_copy(seq_h.at[pl.ds(o, BLK + 16)], b,
                                  seq_sem).wait()
        else:
            @pl.when(islast)
            def _():
                pltpu.make_async_copy(seq_h.at[pl.ds(o, BLK)],
                                      b.at[pl.ds(0, BLK)], seq_sem).wait()
                b[pl.ds(BLK, 16)] = pad

            @pl.when(jnp.logical_not(islast))
            def _():
                pltpu.make_async_copy(seq_h.at[pl.ds(o, BLK + 16)], b,
                                      seq_sem).wait()

    def gather(j):
        return pltpu.make_async_copy(w_sh.at[idxs[j % 2]], vals[j % 2],
                                     gat_sem)

    def merged(j, acc, accumulate):
        b, ij = bufs[j % 2], idxs[j % 2]
        vprev = vals[j % 2]

        @plsc.parallel_loop(0, BLK, step=16, unroll=16, carry=acc)
        def out(i, a3):
            a = b[pl.ds(i, 16)]
            nxt = b[pl.ds(i + 1, 16)]
            ij[pl.ds(i, 16)] = a * NUM_TYPES + nxt
            if accumulate:
                a3 = a3 + vprev[pl.ds(i, 16)]
            return a3

        return out

    def accum_tail2(acc):
        @plsc.parallel_loop(0, BLK, step=16, unroll=16, carry=acc)
        def acc(i, a3):
            return a3 + val0[pl.ds(i, 16)] + val1[pl.ds(i, 16)]

        return acc

    issue_seq(0)
    acc = jnp.zeros((16,), jnp.float32)
    for j in range(NBLK):
        wait_seq(j)
        if j + 1 < NBLK:
            issue_seq(j + 1)
        acc = merged(j, acc, accumulate=(j >= 2))
        if j == 0:
            @pl.when(s == 0)
            def _():
                pltpu.make_async_copy(w_h, w_sh, w_sem).wait()

            plsc.subcore_barrier()
        if j >= 1:
            gather(j - 1).wait()
        gather(j).start()
    gather(NBLK - 1).wait()
    acc = accum_tail2(acc)

    val0[pl.ds(0, 16)] = acc
    pltpu.sync_copy(val0.at[pl.ds(0, 16)], out_h.at[pl.ds(wid * 16, 16)])


@functools.partial(
    pl.kernel,
    out_type=jax.ShapeDtypeStruct((NW * 16,), jnp.float32),
    mesh=plsc.VectorSubcoreMesh(core_axis_name="c", subcore_axis_name="s"),
    scratch_types=[
        pltpu.VMEM_SHARED((ZSLOT,), jnp.float32),
        pltpu.VMEM((BLK + 16,), jnp.int32),
        pltpu.VMEM((BLK + 16,), jnp.int32),
        pltpu.VMEM((BLK,), jnp.int32),
        pltpu.VMEM((BLK,), jnp.int32),
        pltpu.VMEM((BLK,), jnp.float32),
        pltpu.VMEM((BLK,), jnp.float32),
        pltpu.VMEM((16,), jnp.int32),
        pltpu.SemaphoreType.DMA,
        pltpu.SemaphoreType.DMA,
        pltpu.SemaphoreType.DMA,
    ],
)
def _partials(seq_h, w_h, pad_h, out_h, *rest):
    _body(seq_h, w_h, pad_h, out_h, *rest)


def kernel(sequence, padding_idx, W):
    ar = jnp.arange(NUM_TYPES)
    dead = (ar[:, None] == padding_idx) | (ar[None, :] == padding_idx)
    wz = jnp.where(dead, jnp.float32(0), W).reshape(-1)
    padv = jnp.full((16,), padding_idx, dtype=jnp.int32)
    parts = _partials(sequence, wz, padv)
    return -jnp.sum(parts)


# speedup vs baseline: 1.0217x; 1.0214x over previous
import functools

import jax
import jax.numpy as jnp
from jax import lax
from jax.experimental import pallas as pl
from jax.experimental.pallas import tpu as pltpu
from jax.experimental.pallas import tpu_sc as plsc

NUM_TYPES = 1000
SEQ_LEN = 3276800
NC = 2
NS = 16
NW = NC * NS
CHUNK = SEQ_LEN // NW
BLK = 10240
NBLK = CHUNK // BLK
ZSLOT = NUM_TYPES * NUM_TYPES

_DNUMS = lax.GatherDimensionNumbers(
    offset_dims=(), collapsed_slice_dims=(0,), start_index_map=(0,))


def _perm(v, idx):
    return lax.gather(v, idx[:, None], _DNUMS, (1,),
                      mode=lax.GatherScatterMode.PROMISE_IN_BOUNDS)


def _body(seq_h, w_h, pad_h, out_h,
          w_sh, buf0, buf1, idx0, idx1, val0, val1, padv,
          seq_sem, gat_sem, w_sem):
    c = lax.axis_index("c")
    s = lax.axis_index("s")
    wid = s * NC + c
    base = wid * CHUNK
    bufs, idxs, vals = (buf0, buf1), (idx0, idx1), (val0, val1)
    islast = wid == NW - 1

    @pl.when(s == 0)
    def _():
        pltpu.make_async_copy(w_h, w_sh, w_sem).start()

    pltpu.sync_copy(pad_h, padv)
    pad = padv[...]

    def issue_seq(j):
        b = bufs[j % 2]
        o = base + j * BLK
        if j < NBLK - 1:
            pltpu.make_async_copy(seq_h.at[pl.ds(o, BLK + 16)], b,
                                  seq_sem).start()
        else:
            @pl.when(islast)
            def _():
                pltpu.make_async_copy(seq_h.at[pl.ds(o, BLK)],
                                      b.at[pl.ds(0, BLK)], seq_sem).start()

            @pl.when(jnp.logical_not(islast))
            def _():
                pltpu.make_async_copy(seq_h.at[pl.ds(o, BLK + 16)], b,
                                      seq_sem).start()

    def wait_seq(j):
        b = bufs[j % 2]
        o = base + j * BLK
        if j < NBLK - 1:
            pltpu.make_async_copy(seq_h.at[pl.ds(o, BLK + 16)], b,
                                  seq_sem).wait()
        else:
            @pl.when(islast)
            def _():
                pltpu.make_async_copy(seq_h.at[pl.ds(o, BLK)],
                                      b.at[pl.ds(0, BLK)], seq_sem).wait()
                b[pl.ds(BLK, 16)] = pad

            @pl.when(jnp.logical_not(islast))
            def _():
                pltpu.make_async_copy(seq_h.at[pl.ds(o, BLK + 16)], b,
                                      seq_sem).wait()

    def gather(j):
        return pltpu.make_async_copy(w_sh.at[idxs[j % 2]], vals[j % 2],
                                     gat_sem)

    def merged(j, acc, accumulate):
        b, ij = bufs[j % 2], idxs[j % 2]
        vprev = vals[j % 2]

        lanes = lax.iota(jnp.int32, 16)
        rot = (lanes + 1) & 15
        zero16 = lanes * 0
        lane15 = lanes == 15

        @plsc.parallel_loop(0, BLK, step=128, unroll=2, carry=acc)
        def out(i, a3):
            a = [b[pl.ds(i + 16 * k, 16)] for k in range(9)]
            for k in range(8):
                nxt = jnp.where(lane15, _perm(a[k + 1], zero16),
                                _perm(a[k], rot))
                ij[pl.ds(i + 16 * k, 16)] = a[k] * NUM_TYPES + nxt
                if accumulate:
                    a3 = a3 + vprev[pl.ds(i + 16 * k, 16)]
            return a3

        return out

    def accum_tail2(acc):
        @plsc.parallel_loop(0, BLK, step=16, unroll=16, carry=acc)
        def acc(i, a3):
            return a3 + val0[pl.ds(i, 16)] + val1[pl.ds(i, 16)]

        return acc

    issue_seq(0)
    acc = jnp.zeros((16,), jnp.float32)
    for j in range(NBLK):
        wait_seq(j)
        if j + 1 < NBLK:
            issue_seq(j + 1)
        acc = merged(j, acc, accumulate=(j >= 2))
        if j == 0:
            @pl.when(s == 0)
            def _():
                pltpu.make_async_copy(w_h, w_sh, w_sem).wait()

            plsc.subcore_barrier()
        if j >= 1:
            gather(j - 1).wait()
        gather(j).start()
    gather(NBLK - 1).wait()
    acc = accum_tail2(acc)

    val0[pl.ds(0, 16)] = acc
    pltpu.sync_copy(val0.at[pl.ds(0, 16)], out_h.at[pl.ds(wid * 16, 16)])


@functools.partial(
    pl.kernel,
    out_type=jax.ShapeDtypeStruct((NW * 16,), jnp.float32),
    mesh=plsc.VectorSubcoreMesh(core_axis_name="c", subcore_axis_name="s"),
    scratch_types=[
        pltpu.VMEM_SHARED((ZSLOT,), jnp.float32),
        pltpu.VMEM((BLK + 16,), jnp.int32),
        pltpu.VMEM((BLK + 16,), jnp.int32),
        pltpu.VMEM((BLK,), jnp.int32),
        pltpu.VMEM((BLK,), jnp.int32),
        pltpu.VMEM((BLK,), jnp.float32),
        pltpu.VMEM((BLK,), jnp.float32),
        pltpu.VMEM((16,), jnp.int32),
        pltpu.SemaphoreType.DMA,
        pltpu.SemaphoreType.DMA,
        pltpu.SemaphoreType.DMA,
    ],
)
def _partials(seq_h, w_h, pad_h, out_h, *rest):
    _body(seq_h, w_h, pad_h, out_h, *rest)


def kernel(sequence, padding_idx, W):
    ar = jnp.arange(NUM_TYPES)
    dead = (ar[:, None] == padding_idx) | (ar[None, :] == padding_idx)
    wz = jnp.where(dead, jnp.float32(0), W).reshape(-1)
    padv = jnp.full((16,), padding_idx, dtype=jnp.int32)
    parts = _partials(sequence, wz, padv)
    return -jnp.sum(parts)
